# Initial kernel scaffold; baseline (speedup 1.0000x reference)
import jax
import jax.numpy as jnp
from jax.experimental import pallas as pl


def kernel(x, edge_index, W, b):
    raise NotImplementedError("write your pallas kernel here")


# speedup vs baseline: 30.8313x; 30.8313x over previous
import functools

import jax
import jax.numpy as jnp
from jax import lax
from jax.experimental import pallas as pl
from jax.experimental.pallas import tpu as pltpu
from jax.experimental.pallas import tpu_sc as plsc

N_NODES = 10000
NP = 10240
D_FEAT = 128
C = 16
E = 320000
CH = 128
NW = 32
TPC = 79
ECAP = NW * TPC * CH
RPS = NP // 16


def _deg_body(dst_hbm, ones_hbm, zeros_hbm, d_hbm, dst_v, obuf, acc):
    c = lax.axis_index("c")
    s = lax.axis_index("s")
    wid = c * 16 + s
    pltpu.sync_copy(dst_hbm.at[wid], dst_v)
    pltpu.sync_copy(ones_hbm, obuf)
    pltpu.sync_copy(zeros_hbm, acc.at[pl.ds(s * RPS, RPS)])
    plsc.subcore_barrier()

    def body(j, carry):
        pltpu.sync_copy(obuf, acc.at[dst_v.at[j]], add=True)
        return carry

    lax.fori_loop(0, TPC, body, 0)
    plsc.subcore_barrier()
    pltpu.sync_copy(acc.at[pl.ds(s * RPS, RPS)], d_hbm.at[c, pl.ds(s * RPS, RPS)])


def _hop_body(t_hbm, src_hbm, dst_hbm, zeros_hbm, p_hbm,
              src_v, dst_v, gbuf, acc, sem):
    c = lax.axis_index("c")
    s = lax.axis_index("s")
    wid = c * 16 + s
    pltpu.sync_copy(src_hbm.at[wid], src_v)
    pltpu.sync_copy(dst_hbm.at[wid], dst_v)
    pltpu.sync_copy(zeros_hbm, acc.at[pl.ds(s * RPS, RPS)])
    plsc.subcore_barrier()

    def body(j, carry):
        pltpu.async_copy(t_hbm.at[src_v.at[j]], gbuf, sem).wait()
        pltpu.sync_copy(gbuf, acc.at[dst_v.at[j]], add=True)
        return carry

    lax.fori_loop(0, TPC, body, 0)
    plsc.subcore_barrier()
    pltpu.sync_copy(acc.at[pl.ds(s * RPS, RPS)], p_hbm.at[c, pl.ds(s * RPS, RPS)])


_SC_PARAMS = pltpu.CompilerParams(use_tc_tiling_on_sc=False)


def _sc_deg(dst_t, ones_in, zeros_in):
    mesh = plsc.VectorSubcoreMesh(core_axis_name="c", subcore_axis_name="s")
    fn = pl.kernel(
        _deg_body,
        out_type=jax.ShapeDtypeStruct((2, NP, C), jnp.float32),
        mesh=mesh,
        compiler_params=_SC_PARAMS,
        scratch_types=[
            pltpu.VMEM((TPC, CH), jnp.int32),
            pltpu.VMEM((CH, C), jnp.float32),
            pltpu.VMEM_SHARED((NP, C), jnp.float32),
        ],
    )
    return fn(dst_t, ones_in, zeros_in)


def _sc_hop(t, src_t, dst_t, zeros_in):
    mesh = plsc.VectorSubcoreMesh(core_axis_name="c", subcore_axis_name="s")
    fn = pl.kernel(
        _hop_body,
        out_type=jax.ShapeDtypeStruct((2, NP, C), jnp.float32),
        mesh=mesh,
        compiler_params=_SC_PARAMS,
        scratch_types=[
            pltpu.VMEM((TPC, CH), jnp.int32),
            pltpu.VMEM((TPC, CH), jnp.int32),
            pltpu.VMEM((CH, C), jnp.float32),
            pltpu.VMEM_SHARED((NP, C), jnp.float32),
            pltpu.SemaphoreType.DMA,
        ],
    )
    return fn(t, src_t, dst_t, zeros_in)


_BM = 400


def _tc1_kbody(x_ref, w_ref, d0_ref, d1_ref, t1_ref, dv_ref):
    z = lax.dot_general(x_ref[...], w_ref[...], (((1,), (1,)), ((), ())),
                        preferred_element_type=jnp.float32)
    deg = d0_ref[...] + d1_ref[...] + 1.0
    dv = lax.rsqrt(deg)
    dv_ref[...] = dv
    t1_ref[...] = dv * z


def _tc1(x, W, d0, d1):
    return pl.pallas_call(
        _tc1_kbody,
        grid=(N_NODES // _BM,),
        in_specs=[
            pl.BlockSpec((_BM, D_FEAT), lambda i: (i, 0)),
            pl.BlockSpec((C, D_FEAT), lambda i: (0, 0)),
            pl.BlockSpec((_BM, C), lambda i: (i, 0)),
            pl.BlockSpec((_BM, C), lambda i: (i, 0)),
        ],
        out_specs=[
            pl.BlockSpec((_BM, C), lambda i: (i, 0)),
            pl.BlockSpec((_BM, C), lambda i: (i, 0)),
        ],
        out_shape=[
            jax.ShapeDtypeStruct((N_NODES, C), jnp.float32),
            jax.ShapeDtypeStruct((N_NODES, C), jnp.float32),
        ],
    )(x, W, d0, d1)


def _tc2_kbody(p0_ref, p1_ref, t1_ref, dv_ref, o_ref):
    dv = dv_ref[...]
    o_ref[...] = dv * dv * (p0_ref[...] + p1_ref[...] + t1_ref[...])


def _tc2(p0, p1, t1, dv):
    return pl.pallas_call(
        _tc2_kbody,
        out_shape=jax.ShapeDtypeStruct(p0.shape, jnp.float32),
    )(p0, p1, t1, dv)


def _tc3_kbody(q0_ref, q1_ref, t3_ref, dv_ref, bt_ref, o_ref):
    o_ref[...] = (dv_ref[...] * (q0_ref[...] + q1_ref[...] + t3_ref[...])
                  + bt_ref[...])


def _tc3(q0, q1, t3, dv, bt):
    return pl.pallas_call(
        _tc3_kbody,
        out_shape=jax.ShapeDtypeStruct(q0.shape, jnp.float32),
    )(q0, q1, t3, dv, bt)


def kernel(x, edge_index, W, b):
    src = edge_index[0].astype(jnp.int32)
    dst = edge_index[1].astype(jnp.int32)
    pad = jnp.full((ECAP - E,), N_NODES, jnp.int32)
    src_t = jnp.concatenate([src, pad]).reshape(NW, TPC, CH)
    dst_t = jnp.concatenate([dst, pad]).reshape(NW, TPC, CH)
    ones_in = jnp.ones((CH, C), jnp.float32)
    zeros_in = jnp.zeros((RPS, C), jnp.float32)

    dparts = _sc_deg(dst_t, ones_in, zeros_in)
    t1, dv = _tc1(x, W, dparts[0, :N_NODES], dparts[1, :N_NODES])

    zpad = jnp.zeros((NP - N_NODES, C), jnp.float32)
    t1p = jnp.concatenate([t1, zpad])
    dvp = jnp.concatenate([dv, zpad])

    p = _sc_hop(t1p, src_t, dst_t, zeros_in)
    F = NP * C // 128
    dvf = dvp.reshape(F, 128)
    t3f = _tc2(p[0].reshape(F, 128), p[1].reshape(F, 128),
               t1p.reshape(F, 128), dvf)

    q = _sc_hop(t3f.reshape(NP, C), src_t, dst_t, zeros_in)
    bt = jnp.concatenate([b] * 8).reshape(1, 128)
    outf = _tc3(q[0].reshape(F, 128), q[1].reshape(F, 128), t3f, dvf, bt)
    return outf.reshape(NP, C)[:N_NODES]


# speedup vs baseline: 34.3442x; 1.1139x over previous
import functools

import jax
import jax.numpy as jnp
from jax import lax
from jax.experimental import pallas as pl
from jax.experimental.pallas import tpu as pltpu
from jax.experimental.pallas import tpu_sc as plsc

N_NODES = 10000
NP = 10240
D_FEAT = 128
C = 16
E = 320000
CH = 128
NW = 32
TPC = 80
ECAP = NW * TPC * CH
RPS = NP // 16
K = 8
NR = TPC // K


def _deg_body(dst_hbm, ones_hbm, zeros_hbm, d_hbm, dst_v, obuf, acc, ssem):
    c = lax.axis_index("c")
    s = lax.axis_index("s")
    wid = c * 16 + s
    pltpu.sync_copy(dst_hbm.at[wid], dst_v)
    pltpu.sync_copy(ones_hbm, obuf)
    pltpu.sync_copy(zeros_hbm, acc.at[pl.ds(s * RPS, RPS)])
    plsc.subcore_barrier()

    def rnd(r, carry):
        base = r * K
        for m in range(K):
            pltpu.async_copy(obuf, acc.at[dst_v.at[base + m]], ssem, add=True)

        @pl.when(r > 0)
        def _():
            for m in range(K):
                pltpu.make_async_copy(ones_hbm, obuf, ssem).wait()
        return carry

    lax.fori_loop(0, NR, rnd, 0)
    for m in range(K):
        pltpu.make_async_copy(ones_hbm, obuf, ssem).wait()
    plsc.subcore_barrier()
    pltpu.sync_copy(acc.at[pl.ds(s * RPS, RPS)], d_hbm.at[c, pl.ds(s * RPS, RPS)])


def _hop_body(t_hbm, src_hbm, dst_hbm, zeros_hbm, p_hbm,
              src_v, dst_v, gbuf, acc, gsem, ssem):
    c = lax.axis_index("c")
    s = lax.axis_index("s")
    wid = c * 16 + s
    pltpu.sync_copy(src_hbm.at[wid], src_v)
    pltpu.sync_copy(dst_hbm.at[wid], dst_v)
    pltpu.sync_copy(zeros_hbm, acc.at[pl.ds(s * RPS, RPS)])
    plsc.subcore_barrier()

    for m in range(K):
        pltpu.async_copy(t_hbm.at[src_v.at[m]], gbuf.at[m], gsem)

    def rnd(r, carry):
        base = r * K
        off = lax.rem(r, 2) * K
        noff = K - off

        @pl.when(r > 0)
        def _():
            for m in range(K):
                pltpu.make_async_copy(zeros_hbm.at[pl.ds(0, CH)],
                                      gbuf.at[0], ssem).wait()

        for m in range(K):
            j = base + K + m

            @pl.when(j < TPC)
            def _():
                pltpu.async_copy(t_hbm.at[src_v.at[j]], gbuf.at[noff + m], gsem)

        for m in range(K):
            pltpu.make_async_copy(zeros_hbm.at[pl.ds(0, CH)],
                                  gbuf.at[0], gsem).wait()
            pltpu.async_copy(gbuf.at[off + m], acc.at[dst_v.at[base + m]],
                             ssem, add=True)
        return carry

    lax.fori_loop(0, NR, rnd, 0)
    for m in range(K):
        pltpu.make_async_copy(zeros_hbm.at[pl.ds(0, CH)], gbuf.at[0], ssem).wait()
    plsc.subcore_barrier()
    pltpu.sync_copy(acc.at[pl.ds(s * RPS, RPS)], p_hbm.at[c, pl.ds(s * RPS, RPS)])


_SC_PARAMS = pltpu.CompilerParams(use_tc_tiling_on_sc=False)


def _sc_deg(dst_t, ones_in, zeros_in):
    mesh = plsc.VectorSubcoreMesh(core_axis_name="c", subcore_axis_name="s")
    fn = pl.kernel(
        _deg_body,
        out_type=jax.ShapeDtypeStruct((2, NP, C), jnp.float32),
        mesh=mesh,
        compiler_params=_SC_PARAMS,
        scratch_types=[
            pltpu.VMEM((TPC, CH), jnp.int32),
            pltpu.VMEM((CH, C), jnp.float32),
            pltpu.VMEM_SHARED((NP, C), jnp.float32),
            pltpu.SemaphoreType.DMA,
        ],
    )
    return fn(dst_t, ones_in, zeros_in)


def _sc_hop(t, src_t, dst_t, zeros_in):
    mesh = plsc.VectorSubcoreMesh(core_axis_name="c", subcore_axis_name="s")
    fn = pl.kernel(
        _hop_body,
        out_type=jax.ShapeDtypeStruct((2, NP, C), jnp.float32),
        mesh=mesh,
        compiler_params=_SC_PARAMS,
        scratch_types=[
            pltpu.VMEM((TPC, CH), jnp.int32),
            pltpu.VMEM((TPC, CH), jnp.int32),
            pltpu.VMEM((2 * K, CH, C), jnp.float32),
            pltpu.VMEM_SHARED((NP, C), jnp.float32),
            pltpu.SemaphoreType.DMA,
            pltpu.SemaphoreType.DMA,
        ],
    )
    return fn(t, src_t, dst_t, zeros_in)


_BM = 400


def _tc1_kbody(x_ref, w_ref, d0_ref, d1_ref, t1_ref, dv_ref):
    z = lax.dot_general(x_ref[...], w_ref[...], (((1,), (1,)), ((), ())),
                        preferred_element_type=jnp.float32)
    deg = d0_ref[...] + d1_ref[...] + 1.0
    dv = lax.rsqrt(deg)
    dv_ref[...] = dv
    t1_ref[...] = dv * z


def _tc1(x, W, d0, d1):
    return pl.pallas_call(
        _tc1_kbody,
        grid=(N_NODES // _BM,),
        in_specs=[
            pl.BlockSpec((_BM, D_FEAT), lambda i: (i, 0)),
            pl.BlockSpec((C, D_FEAT), lambda i: (0, 0)),
            pl.BlockSpec((_BM, C), lambda i: (i, 0)),
            pl.BlockSpec((_BM, C), lambda i: (i, 0)),
        ],
        out_specs=[
            pl.BlockSpec((_BM, C), lambda i: (i, 0)),
            pl.BlockSpec((_BM, C), lambda i: (i, 0)),
        ],
        out_shape=[
            jax.ShapeDtypeStruct((N_NODES, C), jnp.float32),
            jax.ShapeDtypeStruct((N_NODES, C), jnp.float32),
        ],
    )(x, W, d0, d1)


def _tc2_kbody(p0_ref, p1_ref, t1_ref, dv_ref, o_ref):
    dv = dv_ref[...]
    o_ref[...] = dv * dv * (p0_ref[...] + p1_ref[...] + t1_ref[...])


def _tc2(p0, p1, t1, dv):
    return pl.pallas_call(
        _tc2_kbody,
        out_shape=jax.ShapeDtypeStruct(p0.shape, jnp.float32),
    )(p0, p1, t1, dv)


def _tc3_kbody(q0_ref, q1_ref, t3_ref, dv_ref, bt_ref, o_ref):
    o_ref[...] = (dv_ref[...] * (q0_ref[...] + q1_ref[...] + t3_ref[...])
                  + bt_ref[...])


def _tc3(q0, q1, t3, dv, bt):
    return pl.pallas_call(
        _tc3_kbody,
        out_shape=jax.ShapeDtypeStruct(q0.shape, jnp.float32),
    )(q0, q1, t3, dv, bt)


def kernel(x, edge_index, W, b):
    src = edge_index[0].astype(jnp.int32)
    dst = edge_index[1].astype(jnp.int32)
    pad = jnp.full((ECAP - E,), N_NODES, jnp.int32)
    src_t = jnp.concatenate([src, pad]).reshape(NW, TPC, CH)
    dst_t = jnp.concatenate([dst, pad]).reshape(NW, TPC, CH)
    ones_in = jnp.ones((CH, C), jnp.float32)
    zeros_in = jnp.zeros((RPS, C), jnp.float32)

    dparts = _sc_deg(dst_t, ones_in, zeros_in)
    t1, dv = _tc1(x, W, dparts[0, :N_NODES], dparts[1, :N_NODES])

    zpad = jnp.zeros((NP - N_NODES, C), jnp.float32)
    t1p = jnp.concatenate([t1, zpad])
    dvp = jnp.concatenate([dv, zpad])

    p = _sc_hop(t1p, src_t, dst_t, zeros_in)
    F = NP * C // 128
    dvf = dvp.reshape(F, 128)
    t3f = _tc2(p[0].reshape(F, 128), p[1].reshape(F, 128),
               t1p.reshape(F, 128), dvf)

    q = _sc_hop(t3f.reshape(NP, C), src_t, dst_t, zeros_in)
    bt = jnp.concatenate([b] * 8).reshape(1, 128)
    outf = _tc3(q[0].reshape(F, 128), q[1].reshape(F, 128), t3f, dvf, bt)
    return outf.reshape(NP, C)[:N_NODES]


# speedup vs baseline: 38.6394x; 1.1251x over previous
import jax
import jax.numpy as jnp
from jax import lax
from jax.experimental import pallas as pl
from jax.experimental.pallas import tpu as pltpu
from jax.experimental.pallas import tpu_sc as plsc

N_NODES = 10000
NP = 10240
D_FEAT = 128
C = 16
E = 320000
CH = 128
NCH = 2560
RPS = NP // 16
K = 8

HOP_SPLIT = (40, 120)
DEG_SPLIT = (64, 96)


def _make_deg_body(cnt0, cnt1):
    cmax = max(cnt0, cnt1)
    nr = cmax // K

    def body(e_hbm, ones_hbm, zeros_hbm, d_hbm, dst_v, obuf, acc, ssem):
        c = lax.axis_index("c")
        s = lax.axis_index("s")
        base = jnp.where(c == 0, s * cnt0, 16 * cnt0 + s * cnt1)
        nrc = jnp.where(c == 0, cnt0 // K, cnt1 // K)
        pltpu.sync_copy(e_hbm.at[1, pl.ds(base, cmax)], dst_v)
        pltpu.sync_copy(ones_hbm, obuf)
        pltpu.sync_copy(zeros_hbm, acc.at[pl.ds(s * RPS, RPS)])
        plsc.subcore_barrier()

        def rnd(r, carry):
            @pl.when(r < nrc)
            def _():
                for m in range(K):
                    pltpu.async_copy(obuf, acc.at[dst_v.at[r * K + m]],
                                     ssem, add=True)

            @pl.when((r >= 1) & (r <= nrc))
            def _():
                for m in range(K):
                    pltpu.make_async_copy(ones_hbm, obuf, ssem).wait()
            return carry

        lax.fori_loop(0, nr, rnd, 0)

        @pl.when(nrc == nr)
        def _():
            for m in range(K):
                pltpu.make_async_copy(ones_hbm, obuf, ssem).wait()
        plsc.subcore_barrier()
        pltpu.sync_copy(acc.at[pl.ds(s * RPS, RPS)],
                        d_hbm.at[c, pl.ds(s * RPS, RPS)])

    return body


def _make_hop_body(cnt0, cnt1):
    cmax = max(cnt0, cnt1)
    nr = cmax // K

    def body(t_hbm, e_hbm, zeros_hbm, p_hbm, src_v, dst_v, gbuf, acc,
             gsem, ssem):
        c = lax.axis_index("c")
        s = lax.axis_index("s")
        base = jnp.where(c == 0, s * cnt0, 16 * cnt0 + s * cnt1)
        nrc = jnp.where(c == 0, cnt0 // K, cnt1 // K)
        pltpu.sync_copy(e_hbm.at[0, pl.ds(base, cmax)], src_v)
        pltpu.sync_copy(e_hbm.at[1, pl.ds(base, cmax)], dst_v)
        pltpu.sync_copy(zeros_hbm, acc.at[pl.ds(s * RPS, RPS)])
        plsc.subcore_barrier()

        for m in range(K):
            pltpu.async_copy(t_hbm.at[src_v.at[m]], gbuf.at[m], gsem)

        def rnd(r, carry):
            off = lax.rem(r, 2) * K
            noff = K - off

            @pl.when((r >= 1) & (r <= nrc))
            def _():
                for m in range(K):
                    pltpu.make_async_copy(zeros_hbm.at[pl.ds(0, CH)],
                                          gbuf.at[0], ssem).wait()

            @pl.when(r + 1 < nrc)
            def _():
                for m in range(K):
                    pltpu.async_copy(t_hbm.at[src_v.at[(r + 1) * K + m]],
                                     gbuf.at[noff + m], gsem)

            @pl.when(r < nrc)
            def _():
                for m in range(K):
                    pltpu.make_async_copy(zeros_hbm.at[pl.ds(0, CH)],
                                          gbuf.at[0], gsem).wait()
                    pltpu.async_copy(gbuf.at[off + m],
                                     acc.at[dst_v.at[r * K + m]],
                                     ssem, add=True)
            return carry

        lax.fori_loop(0, nr, rnd, 0)

        @pl.when(nrc == nr)
        def _():
            for m in range(K):
                pltpu.make_async_copy(zeros_hbm.at[pl.ds(0, CH)],
                                      gbuf.at[0], ssem).wait()
        plsc.subcore_barrier()
        pltpu.sync_copy(acc.at[pl.ds(s * RPS, RPS)],
                        p_hbm.at[c, pl.ds(s * RPS, RPS)])

    return body


_SC_PARAMS = pltpu.CompilerParams(use_tc_tiling_on_sc=False)


def _sc_deg(edges, ones_in, zeros_in):
    mesh = plsc.VectorSubcoreMesh(core_axis_name="c", subcore_axis_name="s")
    cmax = max(DEG_SPLIT)
    fn = pl.kernel(
        _make_deg_body(*DEG_SPLIT),
        out_type=jax.ShapeDtypeStruct((2, NP, C), jnp.float32),
        mesh=mesh,
        compiler_params=_SC_PARAMS,
        scratch_types=[
            pltpu.VMEM((cmax, CH), jnp.int32),
            pltpu.VMEM((CH, C), jnp.float32),
            pltpu.VMEM_SHARED((NP, C), jnp.float32),
            pltpu.SemaphoreType.DMA,
        ],
    )
    return fn(edges, ones_in, zeros_in)


def _sc_hop(t, edges, zeros_in):
    mesh = plsc.VectorSubcoreMesh(core_axis_name="c", subcore_axis_name="s")
    cmax = max(HOP_SPLIT)
    fn = pl.kernel(
        _make_hop_body(*HOP_SPLIT),
        out_type=jax.ShapeDtypeStruct((2, NP, C), jnp.float32),
        mesh=mesh,
        compiler_params=_SC_PARAMS,
        scratch_types=[
            pltpu.VMEM((cmax, CH), jnp.int32),
            pltpu.VMEM((cmax, CH), jnp.int32),
            pltpu.VMEM((2 * K, CH, C), jnp.float32),
            pltpu.VMEM_SHARED((NP, C), jnp.float32),
            pltpu.SemaphoreType.DMA,
            pltpu.SemaphoreType.DMA,
        ],
    )
    return fn(t, edges, zeros_in)


def _tc1_kbody(x_ref, w_ref, d_ref, t1_ref, dv_ref):
    z = lax.dot_general(x_ref[...], w_ref[...], (((1,), (1,)), ((), ())),
                        preferred_element_type=jnp.float32)
    deg = d_ref[0, :N_NODES, :] + d_ref[1, :N_NODES, :] + 1.0
    dv = lax.rsqrt(deg)
    zero_tail = jnp.zeros((NP - N_NODES, C), jnp.float32)
    dv_ref[:N_NODES, :] = dv
    dv_ref[N_NODES:, :] = zero_tail
    t1_ref[:N_NODES, :] = dv * z
    t1_ref[N_NODES:, :] = zero_tail


def _tc1(x, W, dparts):
    return pl.pallas_call(
        _tc1_kbody,
        out_shape=[
            jax.ShapeDtypeStruct((NP, C), jnp.float32),
            jax.ShapeDtypeStruct((NP, C), jnp.float32),
        ],
    )(x, W, dparts)


def _tc2_kbody(p_ref, t1_ref, dv_ref, o_ref):
    dv = dv_ref[...]
    o_ref[...] = dv * dv * (p_ref[0] + p_ref[1] + t1_ref[...])


def _tc2(p, t1p, dvp):
    return pl.pallas_call(
        _tc2_kbody,
        out_shape=jax.ShapeDtypeStruct((NP, C), jnp.float32),
    )(p, t1p, dvp)


def _tc3_kbody(q_ref, t3_ref, dv_ref, b_ref, o_ref):
    o_ref[...] = (dv_ref[:N_NODES, :]
                  * (q_ref[0, :N_NODES, :] + q_ref[1, :N_NODES, :]
                     + t3_ref[:N_NODES, :])
                  + b_ref[...])


def _tc3(q, t3p, dvp, b2):
    return pl.pallas_call(
        _tc3_kbody,
        out_shape=jax.ShapeDtypeStruct((N_NODES, C), jnp.float32),
    )(q, t3p, dvp, b2)


def kernel(x, edge_index, W, b):
    ei32 = edge_index.astype(jnp.int32)
    pad = jnp.full((2, NCH * CH - E), N_NODES, jnp.int32)
    edges = jnp.concatenate([ei32, pad], axis=1).reshape(2, NCH, CH)
    ones_in = jnp.ones((CH, C), jnp.float32)
    zeros_in = jnp.zeros((RPS, C), jnp.float32)

    dparts = _sc_deg(edges, ones_in, zeros_in)
    t1p, dvp = _tc1(x, W, dparts)
    p = _sc_hop(t1p, edges, zeros_in)
    t3p = _tc2(p, t1p, dvp)
    q = _sc_hop(t3p, edges, zeros_in)
    return _tc3(q, t3p, dvp, b.reshape(1, C))


# speedup vs baseline: 61.3637x; 1.5881x over previous
import jax
import jax.numpy as jnp
from jax import lax
from jax.experimental import pallas as pl
from jax.experimental.pallas import tpu as pltpu
from jax.experimental.pallas import tpu_sc as plsc

N_NODES = 10000
NP = 10240
D_FEAT = 128
C = 16
E = 320000
CH = 128
NCH = 2560
RPS = NP // 16
K = 8

HOP_SPLIT = (80, 80)
DEG_SPLIT = (96, 64)


def _make_deg_body(cnt0, cnt1):
    cmax = max(cnt0, cnt1)
    nr = cmax // K

    def body(e_hbm, ones_hbm, zeros_hbm, d_hbm, dst_v, obuf, acc, ssem):
        c = lax.axis_index("c")
        s = lax.axis_index("s")
        base = jnp.where(c == 0, s * cnt0, 16 * cnt0 + s * cnt1)
        nrc = jnp.where(c == 0, cnt0 // K, cnt1 // K)
        pltpu.sync_copy(e_hbm.at[1, pl.ds(base, cmax)], dst_v)
        pltpu.sync_copy(ones_hbm, obuf)
        pltpu.sync_copy(zeros_hbm, acc.at[pl.ds(s * RPS, RPS)])
        plsc.subcore_barrier()

        def rnd(r, carry):
            @pl.when(r < nrc)
            def _():
                for m in range(K):
                    pltpu.async_copy(obuf, acc.at[dst_v.at[r * K + m]],
                                     ssem, add=True)

            @pl.when((r >= 1) & (r <= nrc))
            def _():
                for m in range(K):
                    pltpu.make_async_copy(ones_hbm, obuf, ssem).wait()
            return carry

        lax.fori_loop(0, nr, rnd, 0)

        @pl.when(nrc == nr)
        def _():
            for m in range(K):
                pltpu.make_async_copy(ones_hbm, obuf, ssem).wait()
        plsc.subcore_barrier()
        pltpu.sync_copy(acc.at[pl.ds(s * RPS, RPS)],
                        d_hbm.at[c, pl.ds(s * RPS, RPS)])

    return body


def _make_hop_body(cnt0, cnt1):
    cmax = max(cnt0, cnt1)
    nr = cmax // K

    def body(t_hbm, e_hbm, zeros_hbm, p_hbm, src_v, dst_v, gbuf, acc, tspm,
             gsem, ssem):
        c = lax.axis_index("c")
        s = lax.axis_index("s")
        base = jnp.where(c == 0, s * cnt0, 16 * cnt0 + s * cnt1)
        nrc = jnp.where(c == 0, cnt0 // K, cnt1 // K)
        pltpu.sync_copy(e_hbm.at[0, pl.ds(base, cmax)], src_v)
        pltpu.sync_copy(e_hbm.at[1, pl.ds(base, cmax)], dst_v)
        pltpu.sync_copy(zeros_hbm, acc.at[pl.ds(s * RPS, RPS)])
        pltpu.sync_copy(t_hbm.at[pl.ds(s * RPS, RPS)],
                        tspm.at[pl.ds(s * RPS, RPS)])
        plsc.subcore_barrier()

        for m in range(K):
            pltpu.async_copy(tspm.at[src_v.at[m]], gbuf.at[m], gsem)

        def rnd(r, carry):
            off = lax.rem(r, 2) * K
            noff = K - off

            @pl.when((r >= 1) & (r <= nrc))
            def _():
                for m in range(K):
                    pltpu.make_async_copy(zeros_hbm.at[pl.ds(0, CH)],
                                          gbuf.at[0], ssem).wait()

            @pl.when(r + 1 < nrc)
            def _():
                for m in range(K):
                    pltpu.async_copy(tspm.at[src_v.at[(r + 1) * K + m]],
                                     gbuf.at[noff + m], gsem)

            @pl.when(r < nrc)
            def _():
                for m in range(K):
                    pltpu.make_async_copy(zeros_hbm.at[pl.ds(0, CH)],
                                          gbuf.at[0], gsem).wait()
                    pltpu.async_copy(gbuf.at[off + m],
                                     acc.at[dst_v.at[r * K + m]],
                                     ssem, add=True)
            return carry

        lax.fori_loop(0, nr, rnd, 0)

        @pl.when(nrc == nr)
        def _():
            for m in range(K):
                pltpu.make_async_copy(zeros_hbm.at[pl.ds(0, CH)],
                                      gbuf.at[0], ssem).wait()
        plsc.subcore_barrier()
        pltpu.sync_copy(acc.at[pl.ds(s * RPS, RPS)],
                        p_hbm.at[c, pl.ds(s * RPS, RPS)])

    return body


_SC_PARAMS = pltpu.CompilerParams(use_tc_tiling_on_sc=False)


def _sc_deg(edges, ones_in, zeros_in):
    mesh = plsc.VectorSubcoreMesh(core_axis_name="c", subcore_axis_name="s")
    cmax = max(DEG_SPLIT)
    fn = pl.kernel(
        _make_deg_body(*DEG_SPLIT),
        out_type=jax.ShapeDtypeStruct((2, NP, C), jnp.float32),
        mesh=mesh,
        compiler_params=_SC_PARAMS,
        scratch_types=[
            pltpu.VMEM((cmax, CH), jnp.int32),
            pltpu.VMEM((CH, C), jnp.float32),
            pltpu.VMEM_SHARED((NP, C), jnp.float32),
            pltpu.SemaphoreType.DMA,
        ],
    )
    return fn(edges, ones_in, zeros_in)


def _sc_hop(t, edges, zeros_in):
    mesh = plsc.VectorSubcoreMesh(core_axis_name="c", subcore_axis_name="s")
    cmax = max(HOP_SPLIT)
    fn = pl.kernel(
        _make_hop_body(*HOP_SPLIT),
        out_type=jax.ShapeDtypeStruct((2, NP, C), jnp.float32),
        mesh=mesh,
        compiler_params=_SC_PARAMS,
        scratch_types=[
            pltpu.VMEM((cmax, CH), jnp.int32),
            pltpu.VMEM((cmax, CH), jnp.int32),
            pltpu.VMEM((2 * K, CH, C), jnp.float32),
            pltpu.VMEM_SHARED((NP, C), jnp.float32),
            pltpu.VMEM_SHARED((NP, C), jnp.float32),
            pltpu.SemaphoreType.DMA,
            pltpu.SemaphoreType.DMA,
        ],
    )
    return fn(t, edges, zeros_in)


def _tc1_kbody(x_ref, w_ref, d_ref, t1_ref, dv_ref):
    z = lax.dot_general(x_ref[...], w_ref[...], (((1,), (1,)), ((), ())),
                        preferred_element_type=jnp.float32)
    deg = d_ref[0, :N_NODES, :] + d_ref[1, :N_NODES, :] + 1.0
    dv = lax.rsqrt(deg)
    zero_tail = jnp.zeros((NP - N_NODES, C), jnp.float32)
    dv_ref[:N_NODES, :] = dv
    dv_ref[N_NODES:, :] = zero_tail
    t1_ref[:N_NODES, :] = dv * z
    t1_ref[N_NODES:, :] = zero_tail


def _tc1(x, W, dparts):
    return pl.pallas_call(
        _tc1_kbody,
        out_shape=[
            jax.ShapeDtypeStruct((NP, C), jnp.float32),
            jax.ShapeDtypeStruct((NP, C), jnp.float32),
        ],
    )(x, W, dparts)


def _tc2_kbody(p_ref, t1_ref, dv_ref, o_ref):
    dv = dv_ref[...]
    o_ref[...] = dv * dv * (p_ref[0] + p_ref[1] + t1_ref[...])


def _tc2(p, t1p, dvp):
    return pl.pallas_call(
        _tc2_kbody,
        out_shape=jax.ShapeDtypeStruct((NP, C), jnp.float32),
    )(p, t1p, dvp)


def _tc3_kbody(q_ref, t3_ref, dv_ref, b_ref, o_ref):
    o_ref[...] = (dv_ref[:N_NODES, :]
                  * (q_ref[0, :N_NODES, :] + q_ref[1, :N_NODES, :]
                     + t3_ref[:N_NODES, :])
                  + b_ref[...])


def _tc3(q, t3p, dvp, b2):
    return pl.pallas_call(
        _tc3_kbody,
        out_shape=jax.ShapeDtypeStruct((N_NODES, C), jnp.float32),
    )(q, t3p, dvp, b2)


def kernel(x, edge_index, W, b):
    ei32 = edge_index.astype(jnp.int32)
    pad = jnp.full((2, NCH * CH - E), N_NODES, jnp.int32)
    edges = jnp.concatenate([ei32, pad], axis=1).reshape(2, NCH, CH)
    ones_in = jnp.ones((CH, C), jnp.float32)
    zeros_in = jnp.zeros((RPS, C), jnp.float32)

    dparts = _sc_deg(edges, ones_in, zeros_in)
    t1p, dvp = _tc1(x, W, dparts)
    p = _sc_hop(t1p, edges, zeros_in)
    t3p = _tc2(p, t1p, dvp)
    q = _sc_hop(t3p, edges, zeros_in)
    return _tc3(q, t3p, dvp, b.reshape(1, C))


# speedup vs baseline: 89.0498x; 1.4512x over previous
import jax
import jax.numpy as jnp
from jax import lax
from jax.experimental import pallas as pl
from jax.experimental.pallas import tpu as pltpu
from jax.experimental.pallas import tpu_sc as plsc

N_NODES = 10000
NP = 10240
D_FEAT = 128
C = 16
E = 320000
CH = 128
NCH = 2560
RPS = NP // 16
K = 8

HOP_SPLIT = (88, 72)
DEG_SPLIT = (104, 56)

FR = N_NODES * C // 128
FP = NP * C // 128


def _make_deg_body(cnt0, cnt1):
    cmax = max(cnt0, cnt1)
    nr = cmax // K

    def body(e_hbm, ones_hbm, zeros_hbm, d_hbm, dst_v, obuf, acc, ssem):
        c = lax.axis_index("c")
        s = lax.axis_index("s")
        base = jnp.where(c == 0, s * cnt0, 16 * cnt0 + s * cnt1)
        nrc = jnp.where(c == 0, cnt0 // K, cnt1 // K)
        pltpu.sync_copy(e_hbm.at[1, pl.ds(base, cmax)], dst_v)
        pltpu.sync_copy(ones_hbm, obuf)
        pltpu.sync_copy(zeros_hbm, acc.at[pl.ds(s * RPS, RPS)])
        plsc.subcore_barrier()

        def rnd(r, carry):
            @pl.when(r < nrc)
            def _():
                for m in range(K):
                    pltpu.async_copy(obuf, acc.at[dst_v.at[r * K + m]],
                                     ssem, add=True)

            @pl.when((r >= 1) & (r <= nrc))
            def _():
                for m in range(K):
                    pltpu.make_async_copy(ones_hbm, obuf, ssem).wait()
            return carry

        lax.fori_loop(0, nr, rnd, 0)

        @pl.when(nrc == nr)
        def _():
            for m in range(K):
                pltpu.make_async_copy(ones_hbm, obuf, ssem).wait()
        plsc.subcore_barrier()
        pltpu.sync_copy(acc.at[pl.ds(s * RPS, RPS)],
                        d_hbm.at[c, pl.ds(s * RPS, RPS)])

    return body


def _make_hop_body(cnt0, cnt1):
    cmax = max(cnt0, cnt1)
    nr = cmax // K

    def body(t_hbm, e_hbm, zeros_hbm, p_hbm, src_v, dst_v, gbuf, acc, tspm,
             gsem, ssem):
        c = lax.axis_index("c")
        s = lax.axis_index("s")
        base = jnp.where(c == 0, s * cnt0, 16 * cnt0 + s * cnt1)
        nrc = jnp.where(c == 0, cnt0 // K, cnt1 // K)
        pltpu.sync_copy(e_hbm.at[0, pl.ds(base, cmax)], src_v)
        pltpu.sync_copy(e_hbm.at[1, pl.ds(base, cmax)], dst_v)
        pltpu.sync_copy(zeros_hbm, acc.at[pl.ds(s * RPS, RPS)])
        pltpu.sync_copy(t_hbm.at[pl.ds(s * RPS, RPS)],
                        tspm.at[pl.ds(s * RPS, RPS)])
        plsc.subcore_barrier()

        for m in range(K):
            pltpu.async_copy(tspm.at[src_v.at[m]], gbuf.at[m], gsem)

        def rnd(r, carry):
            off = lax.rem(r, 2) * K
            noff = K - off

            @pl.when((r >= 1) & (r <= nrc))
            def _():
                for m in range(K):
                    pltpu.make_async_copy(zeros_hbm.at[pl.ds(0, CH)],
                                          gbuf.at[0], ssem).wait()

            @pl.when(r + 1 < nrc)
            def _():
                for m in range(K):
                    pltpu.async_copy(tspm.at[src_v.at[(r + 1) * K + m]],
                                     gbuf.at[noff + m], gsem)

            @pl.when(r < nrc)
            def _():
                for m in range(K):
                    pltpu.make_async_copy(zeros_hbm.at[pl.ds(0, CH)],
                                          gbuf.at[0], gsem).wait()
                    pltpu.async_copy(gbuf.at[off + m],
                                     acc.at[dst_v.at[r * K + m]],
                                     ssem, add=True)
            return carry

        lax.fori_loop(0, nr, rnd, 0)

        @pl.when(nrc == nr)
        def _():
            for m in range(K):
                pltpu.make_async_copy(zeros_hbm.at[pl.ds(0, CH)],
                                      gbuf.at[0], ssem).wait()
        plsc.subcore_barrier()
        pltpu.sync_copy(acc.at[pl.ds(s * RPS, RPS)],
                        p_hbm.at[c, pl.ds(s * RPS, RPS)])

    return body


_SC_PARAMS = pltpu.CompilerParams(use_tc_tiling_on_sc=False)


def _sc_deg(edges, ones_in, zeros_in):
    mesh = plsc.VectorSubcoreMesh(core_axis_name="c", subcore_axis_name="s")
    cmax = max(DEG_SPLIT)
    fn = pl.kernel(
        _make_deg_body(*DEG_SPLIT),
        out_type=jax.ShapeDtypeStruct((2, NP, C), jnp.float32),
        mesh=mesh,
        compiler_params=_SC_PARAMS,
        scratch_types=[
            pltpu.VMEM((cmax, CH), jnp.int32),
            pltpu.VMEM((CH, C), jnp.float32),
            pltpu.VMEM_SHARED((NP, C), jnp.float32),
            pltpu.SemaphoreType.DMA,
        ],
    )
    return fn(edges, ones_in, zeros_in)


def _sc_hop(t, edges, zeros_in):
    mesh = plsc.VectorSubcoreMesh(core_axis_name="c", subcore_axis_name="s")
    cmax = max(HOP_SPLIT)
    fn = pl.kernel(
        _make_hop_body(*HOP_SPLIT),
        out_type=jax.ShapeDtypeStruct((2, NP, C), jnp.float32),
        mesh=mesh,
        compiler_params=_SC_PARAMS,
        scratch_types=[
            pltpu.VMEM((cmax, CH), jnp.int32),
            pltpu.VMEM((cmax, CH), jnp.int32),
            pltpu.VMEM((2 * K, CH, C), jnp.float32),
            pltpu.VMEM_SHARED((NP, C), jnp.float32),
            pltpu.VMEM_SHARED((NP, C), jnp.float32),
            pltpu.SemaphoreType.DMA,
            pltpu.SemaphoreType.DMA,
        ],
    )
    return fn(t, edges, zeros_in)


def _tc1_kbody(xp_ref, wb_ref, d_ref, t1_ref, dv_ref):
    z = lax.dot_general(xp_ref[...], wb_ref[...], (((1,), (0,)), ((), ())),
                        preferred_element_type=jnp.float32)
    deg = d_ref[0, :FR, :] + d_ref[1, :FR, :] + 1.0
    dv = lax.rsqrt(deg)
    zero_tail = jnp.zeros((FP - FR, 128), jnp.float32)
    dv_ref[:FR, :] = dv
    dv_ref[FR:, :] = zero_tail
    t1_ref[:FR, :] = dv * z
    t1_ref[FR:, :] = zero_tail


def _tc1(xp, wb, dpacked):
    return pl.pallas_call(
        _tc1_kbody,
        out_shape=[
            jax.ShapeDtypeStruct((FP, 128), jnp.float32),
            jax.ShapeDtypeStruct((FP, 128), jnp.float32),
        ],
    )(xp, wb, dpacked)


def _tc2_kbody(p_ref, t1_ref, dv_ref, o_ref):
    dv = dv_ref[...]
    o_ref[...] = dv * dv * (p_ref[0] + p_ref[1] + t1_ref[...])


def _tc2(p, t1p, dvp):
    return pl.pallas_call(
        _tc2_kbody,
        out_shape=jax.ShapeDtypeStruct((FP, 128), jnp.float32),
    )(p, t1p, dvp)


def _tc3_kbody(q_ref, t3_ref, dv_ref, b_ref, o_ref):
    o_ref[...] = (dv_ref[:FR, :]
                  * (q_ref[0, :FR, :] + q_ref[1, :FR, :] + t3_ref[:FR, :])
                  + b_ref[...])


def _tc3(q, t3p, dvp, bt):
    return pl.pallas_call(
        _tc3_kbody,
        out_shape=jax.ShapeDtypeStruct((FR, 128), jnp.float32),
    )(q, t3p, dvp, bt)


def kernel(x, edge_index, W, b):
    ei32 = edge_index.astype(jnp.int32)
    pad = jnp.full((2, NCH * CH - E), N_NODES, jnp.int32)
    edges = jnp.concatenate([ei32, pad], axis=1).reshape(2, NCH, CH)
    ones_in = jnp.ones((CH, C), jnp.float32)
    zeros_in = jnp.zeros((RPS, C), jnp.float32)

    xp = x.reshape(FR, 8 * D_FEAT)
    wb = jnp.kron(jnp.eye(8, dtype=jnp.float32), W.T)
    bt = jnp.tile(b, 8).reshape(1, 128)

    dparts = _sc_deg(edges, ones_in, zeros_in)
    t1f, dvf = _tc1(xp, wb, dparts.reshape(2, FP, 128))
    p = _sc_hop(t1f.reshape(NP, C), edges, zeros_in)
    t3f = _tc2(p.reshape(2, FP, 128), t1f, dvf)
    q = _sc_hop(t3f.reshape(NP, C), edges, zeros_in)
    outf = _tc3(q.reshape(2, FP, 128), t3f, dvf, bt)
    return outf.reshape(N_NODES, C)
